# baseline (speedup 1.0000x reference)
import functools

import jax
import jax.numpy as jnp
from jax import lax
from jax.experimental import pallas as pl
from jax.experimental.pallas import tpu as pltpu
import jax.experimental.pallas.tpu_sc as plsc

NC, NS, L = 2, 16, 16
NW = NC * NS
N_P = 10000
N_A = 10000
D = 128
E = 320000
E_LABEL = 100000

CH = 128
CPT = 160
E_PAD = NS * CPT * CH
DUMMY = N_A
ACC_R = N_A + 8
APW = 624
TAIL = N_A - NS * APW
DH = D // NC
K = 2
G = CPT // K
HIST_R = N_A + L

LCH = 26
EL_PAD = NW * LCH * CH


def _zero_f32(ref, rows, cols):
    def body(t, _):
        r = t // (cols // L)
        c = (t % (cols // L)) * L
        ref[r, pl.ds(c, L)] = jnp.zeros((L,), jnp.float32)
        return 0
    lax.fori_loop(0, rows * (cols // L), body, 0)


def _scatter_body(aid_hbm, pid_hbm, px_hbm, sum_hbm, cnt_hbm,
                  aid_v, pid_v, rows_v, hist_v, acc_sh, gsem, ssem):
    cid = lax.axis_index("c")
    sid = lax.axis_index("s")
    wid = cid * NS + sid

    BW = K * CH
    _zero_f32(rows_v.at[0], BW, DH)
    def zh(t, _):
        hist_v[pl.ds(t * L, L)] = jnp.zeros((L,), jnp.float32)
        return 0
    lax.fori_loop(0, HIST_R // L, zh, 0)
    r0 = sid * APW
    nt = APW - (APW // BW) * BW
    for k in range(APW // BW):
        pltpu.sync_copy(rows_v.at[0], acc_sh.at[pl.ds(r0 + k * BW, BW)])
    pltpu.sync_copy(rows_v.at[0].at[pl.ds(0, nt)],
                    acc_sh.at[pl.ds(r0 + (APW // BW) * BW, nt)])
    @pl.when(sid == NS - 1)
    def _():
        pltpu.sync_copy(rows_v.at[0].at[pl.ds(0, TAIL)],
                        acc_sh.at[pl.ds(NS * APW, TAIL)])
    plsc.subcore_barrier()

    pltpu.sync_copy(aid_hbm.at[sid], aid_v)
    pltpu.sync_copy(pid_hbm.at[sid], pid_v)

    vone = jnp.ones((L,), jnp.float32)

    def issue_gathers(g, p):
        pltpu.async_copy(px_hbm.at[cid].at[pid_v.at[g]], rows_v.at[p], gsem)

    def wait_gathers(g, p):
        pltpu.make_async_copy(px_hbm.at[cid].at[pid_v.at[g]],
                              rows_v.at[p], gsem).wait()

    def issue_scatters(g, p):
        pltpu.async_copy(rows_v.at[p],
                         acc_sh.at[aid_v.at[g]], ssem, add=True)

    def wait_scatters(g, p):
        pltpu.make_async_copy(rows_v.at[p],
                              acc_sh.at[aid_v.at[g]], ssem).wait()

    def histogram(g):
        for k2 in range(BW // L):
            idx = aid_v[g, pl.ds(k2 * L, L)]
            plsc.addupdate_scatter(hist_v, [idx], vone)

    issue_gathers(0, 0)

    def super_group(t, _):
        for p in range(2):
            g = 2 * t + p
            wait_gathers(g, p)
            issue_scatters(g, p)
            @pl.when(g + 1 < G)
            def _():
                issue_gathers(g + 1, 1 - p)
            histogram(g)
            wait_scatters(g, p)
        return 0

    lax.fori_loop(0, G // 2, super_group, 0)

    plsc.subcore_barrier()

    pltpu.sync_copy(acc_sh.at[pl.ds(r0, APW)], sum_hbm.at[cid, pl.ds(r0, APW)])
    @pl.when(sid == NS - 1)
    def _():
        pltpu.sync_copy(acc_sh.at[pl.ds(NS * APW, TAIL)],
                        sum_hbm.at[cid, pl.ds(NS * APW, TAIL)])
    pltpu.sync_copy(hist_v.at[pl.ds(0, N_A)], cnt_hbm.at[wid])


@functools.cache
def _scatter_call():
    mesh = plsc.VectorSubcoreMesh(
        core_axis_name="c", subcore_axis_name="s",
        num_cores=NC, num_subcores=NS)
    return pl.kernel(
        _scatter_body,
        out_type=(
            jax.ShapeDtypeStruct((NC, N_A, DH), jnp.float32),
            jax.ShapeDtypeStruct((NW, N_A), jnp.float32),
        ),
        mesh=mesh,
        compiler_params=pltpu.CompilerParams(
            use_tc_tiling_on_sc=False, needs_layout_passes=False),
        scratch_types=[
            pltpu.VMEM((G, K * CH), jnp.int32),
            pltpu.VMEM((G, K * CH), jnp.int32),
            pltpu.VMEM((2, K * CH, DH), jnp.float32),
            pltpu.VMEM((HIST_R,), jnp.float32),
            pltpu.VMEM_SHARED((ACC_R, DH), jnp.float32),
            pltpu.SemaphoreType.DMA,
            pltpu.SemaphoreType.DMA,
        ],
    )


def _dense_body(px_ref, s_ref, c_ref, wp_ref, bp_ref, wa_ref, ba_ref,
                ph_ref, ah_ref):
    dn = (((1,), (1,)), ((), ()))
    wp = wp_ref[...]
    bp = bp_ref[...]
    px = px_ref[...]
    ph_ref[...] = lax.dot_general(
        px, wp, dn, precision=lax.Precision.HIGHEST,
        preferred_element_type=jnp.float32) + bp
    s = jnp.concatenate([s_ref[0], s_ref[1]], axis=1)
    cnt = 0.5 * jnp.sum(c_ref[...], axis=1)[:, None]
    mean = s / jnp.maximum(cnt, 1.0)
    t = lax.dot_general(mean, wp, dn, precision=lax.Precision.HIGHEST,
                        preferred_element_type=jnp.float32)
    am = t + bp * (cnt > 0.0).astype(jnp.float32)
    ah_ref[...] = lax.dot_general(
        am, wa_ref[...], dn, precision=lax.Precision.HIGHEST,
        preferred_element_type=jnp.float32) + ba_ref[...]


_BLK = 1000

_dense_call = pl.pallas_call(
    _dense_body,
    grid=(N_P // _BLK,),
    in_specs=[
        pl.BlockSpec((_BLK, D), lambda i: (i, 0)),
        pl.BlockSpec((NC, _BLK, DH), lambda i: (0, i, 0)),
        pl.BlockSpec((_BLK, NW), lambda i: (i, 0)),
        pl.BlockSpec((D, D), lambda i: (0, 0)),
        pl.BlockSpec((1, D), lambda i: (0, 0)),
        pl.BlockSpec((D, D), lambda i: (0, 0)),
        pl.BlockSpec((1, D), lambda i: (0, 0)),
    ],
    out_specs=[
        pl.BlockSpec((_BLK, D), lambda i: (i, 0)),
        pl.BlockSpec((_BLK, D), lambda i: (i, 0)),
    ],
    out_shape=[
        jax.ShapeDtypeStruct((N_P, D), jnp.float32),
        jax.ShapeDtypeStruct((N_A, D), jnp.float32),
    ],
)


def _pred_body(row_hbm, col_hbm, ah_hbm, ph_hbm, pred_hbm,
               r_v, c_v, ar_v, pr_v, out_v, tmp_v, sem):
    cid = lax.axis_index("c")
    sid = lax.axis_index("s")
    wid = cid * NS + sid

    pltpu.sync_copy(row_hbm.at[wid], r_v)
    pltpu.sync_copy(col_hbm.at[wid], c_v)

    lane = lax.iota(jnp.int32, L)

    def issue(ci, b):
        pltpu.async_copy(ah_hbm.at[r_v.at[ci]], ar_v.at[b], sem)
        pltpu.async_copy(ph_hbm.at[c_v.at[ci]], pr_v.at[b], sem)

    def wait(ci, b):
        pltpu.make_async_copy(ah_hbm.at[r_v.at[ci]], ar_v.at[b], sem).wait()
        pltpu.make_async_copy(ph_hbm.at[c_v.at[ci]], pr_v.at[b], sem).wait()

    def compute(ci, b):
        def gbody(g, _):
            for u in range(L):
                e = g * L + u
                acc = (ar_v[b, e, pl.ds(0, L)] * pr_v[b, e, pl.ds(0, L)])
                for j in range(1, D // L):
                    acc = acc + (ar_v[b, e, pl.ds(j * L, L)]
                                 * pr_v[b, e, pl.ds(j * L, L)])
                tmp_v[u, pl.ds(0, L)] = acc
            res = plsc.load_gather(tmp_v, [lane, jnp.zeros((L,), jnp.int32)])
            for c in range(1, L):
                res = res + plsc.load_gather(
                    tmp_v, [lane, jnp.full((L,), c, jnp.int32)])
            out_v[ci, pl.ds(g * L, L)] = res
            return 0
        lax.fori_loop(0, CH // L, gbody, 0)

    issue(0, 0)

    def super_chunk(t, _):
        for b in range(2):
            ci = 2 * t + b
            wait(ci, b)
            @pl.when(ci + 1 < LCH)
            def _():
                issue(ci + 1, 1 - b)
            compute(ci, b)
        return 0

    lax.fori_loop(0, LCH // 2, super_chunk, 0)
    pltpu.sync_copy(out_v, pred_hbm.at[wid])


@functools.cache
def _pred_call():
    mesh = plsc.VectorSubcoreMesh(
        core_axis_name="c", subcore_axis_name="s",
        num_cores=NC, num_subcores=NS)
    return pl.kernel(
        _pred_body,
        out_type=jax.ShapeDtypeStruct((NW, LCH, CH), jnp.float32),
        mesh=mesh,
        compiler_params=pltpu.CompilerParams(needs_layout_passes=False),
        scratch_types=[
            pltpu.VMEM((LCH, CH), jnp.int32),
            pltpu.VMEM((LCH, CH), jnp.int32),
            pltpu.VMEM((2, CH, D), jnp.float32),
            pltpu.VMEM((2, CH, D), jnp.float32),
            pltpu.VMEM((LCH, CH), jnp.float32),
            pltpu.VMEM((L, L), jnp.float32),
            pltpu.SemaphoreType.DMA,
        ],
    )


def kernel(paper_x, edge_index, edge_label_index, Wp, bp, Wa, ba):
    epad = E_PAD - E
    aid = jnp.concatenate(
        [edge_index[0], jnp.full((epad,), DUMMY, jnp.int32)]
    ).reshape(NS, G, K * CH)
    pid = jnp.concatenate(
        [edge_index[1], jnp.zeros((epad,), jnp.int32)]
    ).reshape(NS, G, K * CH)
    px_halves = paper_x.reshape(N_P, NC, DH).transpose(1, 0, 2)
    sums, cnts = _scatter_call()(aid, pid, px_halves)
    ph, ah = _dense_call(paper_x, sums, cnts.T, Wp, bp.reshape(1, D),
                         Wa, ba.reshape(1, D))
    pad = EL_PAD - E_LABEL
    row = jnp.concatenate(
        [edge_label_index[0], jnp.zeros((pad,), jnp.int32)]
    ).reshape(NW, LCH, CH)
    col = jnp.concatenate(
        [edge_label_index[1], jnp.zeros((pad,), jnp.int32)]
    ).reshape(NW, LCH, CH)
    pred = _pred_call()(row, col, ah, ph)
    return pred.reshape(-1)[:E_LABEL]


# speedup vs baseline: 1.6516x; 1.6516x over previous
import functools

import jax
import jax.numpy as jnp
from jax import lax
from jax.experimental import pallas as pl
from jax.experimental.pallas import tpu as pltpu
import jax.experimental.pallas.tpu_sc as plsc

NC, NS, L = 2, 16, 16
NW = NC * NS
N_P = 10000
N_A = 10000
D = 128
E = 320000
E_LABEL = 100000

CH = 128
CPT = 160
E_PAD = NS * CPT * CH
DUMMY = N_A
ACC_R = N_A + 8
APW = 624
TAIL = N_A - NS * APW
DH = D // NC
K = 2
G = CPT // K
HIST_R = N_A + L

LCH = 26
EL_PAD = NW * LCH * CH


def _zero_f32(ref, rows, cols):
    def body(t, _):
        r = t // (cols // L)
        c = (t % (cols // L)) * L
        ref[r, pl.ds(c, L)] = jnp.zeros((L,), jnp.float32)
        return 0
    lax.fori_loop(0, rows * (cols // L), body, 0)


def _scatter_body(aid_hbm, pid_hbm, px_hbm, sum_hbm, cnt_hbm,
                  aid_v, pid_v, rows_v, hist_v, acc_sh, gsem, ssem):
    cid = lax.axis_index("c")
    sid = lax.axis_index("s")
    wid = cid * NS + sid

    BW = K * CH
    _zero_f32(rows_v.at[0], BW, DH)
    def zh(t, _):
        hist_v[pl.ds(t * L, L)] = jnp.zeros((L,), jnp.float32)
        return 0
    lax.fori_loop(0, HIST_R // L, zh, 0)
    r0 = sid * APW
    nt = APW - (APW // BW) * BW
    for k in range(APW // BW):
        pltpu.sync_copy(rows_v.at[0], acc_sh.at[pl.ds(r0 + k * BW, BW)])
    pltpu.sync_copy(rows_v.at[0].at[pl.ds(0, nt)],
                    acc_sh.at[pl.ds(r0 + (APW // BW) * BW, nt)])
    @pl.when(sid == NS - 1)
    def _():
        pltpu.sync_copy(rows_v.at[0].at[pl.ds(0, TAIL)],
                        acc_sh.at[pl.ds(NS * APW, TAIL)])
    plsc.subcore_barrier()

    pltpu.sync_copy(aid_hbm.at[sid], aid_v)
    pltpu.sync_copy(pid_hbm.at[sid], pid_v)

    vone = jnp.ones((L,), jnp.float32)

    def issue_gathers(g, p):
        pltpu.async_copy(px_hbm.at[cid].at[pid_v.at[g]], rows_v.at[p], gsem)

    def wait_gathers(g, p):
        pltpu.make_async_copy(px_hbm.at[cid].at[pid_v.at[g]],
                              rows_v.at[p], gsem).wait()

    def issue_scatters(g, p):
        pltpu.async_copy(rows_v.at[p],
                         acc_sh.at[aid_v.at[g]], ssem, add=True)

    def wait_scatters(g, p):
        pltpu.make_async_copy(rows_v.at[p],
                              acc_sh.at[aid_v.at[g]], ssem).wait()

    def histogram(g):
        for k2 in range(BW // L):
            idx = aid_v[g, pl.ds(k2 * L, L)]
            plsc.addupdate_scatter(hist_v, [idx], vone)

    issue_gathers(0, 0)

    def super_group(t, _):
        for p in range(2):
            g = 2 * t + p
            wait_gathers(g, p)
            issue_scatters(g, p)
            @pl.when(g + 1 < G)
            def _():
                issue_gathers(g + 1, 1 - p)
            histogram(g)
            wait_scatters(g, p)
        return 0

    lax.fori_loop(0, G // 2, super_group, 0)

    plsc.subcore_barrier()

    pltpu.sync_copy(acc_sh.at[pl.ds(r0, APW)], sum_hbm.at[cid, pl.ds(r0, APW)])
    @pl.when(sid == NS - 1)
    def _():
        pltpu.sync_copy(acc_sh.at[pl.ds(NS * APW, TAIL)],
                        sum_hbm.at[cid, pl.ds(NS * APW, TAIL)])
    pltpu.sync_copy(hist_v.at[pl.ds(0, N_A)], cnt_hbm.at[wid])


@functools.cache
def _scatter_call():
    mesh = plsc.VectorSubcoreMesh(
        core_axis_name="c", subcore_axis_name="s",
        num_cores=NC, num_subcores=NS)
    return pl.kernel(
        _scatter_body,
        out_type=(
            jax.ShapeDtypeStruct((NC, N_A, DH), jnp.float32),
            jax.ShapeDtypeStruct((NW, N_A), jnp.float32),
        ),
        mesh=mesh,
        compiler_params=pltpu.CompilerParams(
            use_tc_tiling_on_sc=False, needs_layout_passes=False),
        scratch_types=[
            pltpu.VMEM((G, K * CH), jnp.int32),
            pltpu.VMEM((G, K * CH), jnp.int32),
            pltpu.VMEM((2, K * CH, DH), jnp.float32),
            pltpu.VMEM((HIST_R,), jnp.float32),
            pltpu.VMEM_SHARED((ACC_R, DH), jnp.float32),
            pltpu.SemaphoreType.DMA,
            pltpu.SemaphoreType.DMA,
        ],
    )


def _dense_body(px_ref, s_ref, c_ref, wp_ref, bp_ref, wa_ref, ba_ref,
                ph_ref, ah_ref):
    dn = (((1,), (1,)), ((), ()))
    wp = wp_ref[...]
    bp = bp_ref[...]
    px = px_ref[...]
    ph = lax.dot_general(
        px, wp, dn, precision=lax.Precision.HIGHEST,
        preferred_element_type=jnp.float32) + bp
    ph_ref[...] = ph.astype(jnp.bfloat16)
    s = jnp.concatenate([s_ref[0], s_ref[1]], axis=1)
    cnt = 0.5 * jnp.sum(c_ref[...], axis=1)[:, None]
    mean = s / jnp.maximum(cnt, 1.0)
    t = lax.dot_general(mean, wp, dn, precision=lax.Precision.HIGHEST,
                        preferred_element_type=jnp.float32)
    am = t + bp * (cnt > 0.0).astype(jnp.float32)
    ah = lax.dot_general(
        am, wa_ref[...], dn, precision=lax.Precision.HIGHEST,
        preferred_element_type=jnp.float32) + ba_ref[...]
    ah_ref[...] = ah.astype(jnp.bfloat16)


_BLK = 1000

_dense_call = pl.pallas_call(
    _dense_body,
    grid=(N_P // _BLK,),
    in_specs=[
        pl.BlockSpec((_BLK, D), lambda i: (i, 0)),
        pl.BlockSpec((NC, _BLK, DH), lambda i: (0, i, 0)),
        pl.BlockSpec((_BLK, NW), lambda i: (i, 0)),
        pl.BlockSpec((D, D), lambda i: (0, 0)),
        pl.BlockSpec((1, D), lambda i: (0, 0)),
        pl.BlockSpec((D, D), lambda i: (0, 0)),
        pl.BlockSpec((1, D), lambda i: (0, 0)),
    ],
    out_specs=[
        pl.BlockSpec((_BLK, D), lambda i: (i, 0)),
        pl.BlockSpec((_BLK, D), lambda i: (i, 0)),
    ],
    out_shape=[
        jax.ShapeDtypeStruct((N_P, D), jnp.bfloat16),
        jax.ShapeDtypeStruct((N_A, D), jnp.bfloat16),
    ],
)


def _pred_body(row_hbm, col_hbm, ah_hbm, ph_hbm, pred_hbm,
               r_v, c_v, ar_v, pr_v, out_v, tmp_v, ah_sh, ph_sh, sem):
    cid = lax.axis_index("c")
    sid = lax.axis_index("s")
    wid = cid * NS + sid

    r0 = sid * APW
    pltpu.sync_copy(ah_hbm.at[pl.ds(r0, APW)], ah_sh.at[pl.ds(r0, APW)])
    pltpu.sync_copy(ph_hbm.at[pl.ds(r0, APW)], ph_sh.at[pl.ds(r0, APW)])
    @pl.when(sid == NS - 1)
    def _():
        pltpu.sync_copy(ah_hbm.at[pl.ds(NS * APW, TAIL)],
                        ah_sh.at[pl.ds(NS * APW, TAIL)])
        pltpu.sync_copy(ph_hbm.at[pl.ds(NS * APW, TAIL)],
                        ph_sh.at[pl.ds(NS * APW, TAIL)])

    pltpu.sync_copy(row_hbm.at[wid], r_v)
    pltpu.sync_copy(col_hbm.at[wid], c_v)
    plsc.subcore_barrier()

    lane = lax.iota(jnp.int32, L)

    def issue(ci, b):
        pltpu.async_copy(ah_sh.at[r_v.at[ci]], ar_v.at[b], sem)
        pltpu.async_copy(ph_sh.at[c_v.at[ci]], pr_v.at[b], sem)

    def wait(ci, b):
        pltpu.make_async_copy(ah_sh.at[r_v.at[ci]], ar_v.at[b], sem).wait()
        pltpu.make_async_copy(ph_sh.at[c_v.at[ci]], pr_v.at[b], sem).wait()

    def compute(ci, b):
        def gbody(g, _):
            for u in range(L):
                e = g * L + u
                acc = jnp.zeros((L,), jnp.float32)
                for j in range(D // (2 * L)):
                    a2 = ar_v[b, e, pl.ds(j * 2 * L, 2 * L)]
                    p2 = pr_v[b, e, pl.ds(j * 2 * L, 2 * L)]
                    a0, a1 = plsc.unpack(a2, format=plsc.PackFormat.INTERLEAVED)
                    p0, p1 = plsc.unpack(p2, format=plsc.PackFormat.INTERLEAVED)
                    acc = acc + a0 * p0 + a1 * p1
                tmp_v[u, pl.ds(0, L)] = acc
            res = plsc.load_gather(tmp_v, [lane, jnp.zeros((L,), jnp.int32)])
            for c in range(1, L):
                res = res + plsc.load_gather(
                    tmp_v, [lane, jnp.full((L,), c, jnp.int32)])
            out_v[ci, pl.ds(g * L, L)] = res
            return 0
        lax.fori_loop(0, CH // L, gbody, 0)

    issue(0, 0)

    def super_chunk(t, _):
        for b in range(2):
            ci = 2 * t + b
            wait(ci, b)
            @pl.when(ci + 1 < LCH)
            def _():
                issue(ci + 1, 1 - b)
            compute(ci, b)
        return 0

    lax.fori_loop(0, LCH // 2, super_chunk, 0)
    pltpu.sync_copy(out_v, pred_hbm.at[wid])


@functools.cache
def _pred_call():
    mesh = plsc.VectorSubcoreMesh(
        core_axis_name="c", subcore_axis_name="s",
        num_cores=NC, num_subcores=NS)
    return pl.kernel(
        _pred_body,
        out_type=jax.ShapeDtypeStruct((NW, LCH, CH), jnp.float32),
        mesh=mesh,
        compiler_params=pltpu.CompilerParams(
            use_tc_tiling_on_sc=False, needs_layout_passes=False),
        scratch_types=[
            pltpu.VMEM((LCH, CH), jnp.int32),
            pltpu.VMEM((LCH, CH), jnp.int32),
            pltpu.VMEM((2, CH, D), jnp.bfloat16),
            pltpu.VMEM((2, CH, D), jnp.bfloat16),
            pltpu.VMEM((LCH, CH), jnp.float32),
            pltpu.VMEM((L, L), jnp.float32),
            pltpu.VMEM_SHARED((N_A, D), jnp.bfloat16),
            pltpu.VMEM_SHARED((N_A, D), jnp.bfloat16),
            pltpu.SemaphoreType.DMA,
        ],
    )


def kernel(paper_x, edge_index, edge_label_index, Wp, bp, Wa, ba):
    epad = E_PAD - E
    aid = jnp.concatenate(
        [edge_index[0], jnp.full((epad,), DUMMY, jnp.int32)]
    ).reshape(NS, G, K * CH)
    pid = jnp.concatenate(
        [edge_index[1], jnp.zeros((epad,), jnp.int32)]
    ).reshape(NS, G, K * CH)
    px_halves = paper_x.reshape(N_P, NC, DH).transpose(1, 0, 2)
    sums, cnts = _scatter_call()(aid, pid, px_halves)
    ph, ah = _dense_call(paper_x, sums, cnts.T, Wp, bp.reshape(1, D),
                         Wa, ba.reshape(1, D))
    pad = EL_PAD - E_LABEL
    row = jnp.concatenate(
        [edge_label_index[0], jnp.zeros((pad,), jnp.int32)]
    ).reshape(NW, LCH, CH)
    col = jnp.concatenate(
        [edge_label_index[1], jnp.zeros((pad,), jnp.int32)]
    ).reshape(NW, LCH, CH)
    pred = _pred_call()(row, col, ah, ph)
    return pred.reshape(-1)[:E_LABEL]


# speedup vs baseline: 2.4188x; 1.4645x over previous
import functools

import jax
import jax.numpy as jnp
from jax import lax
from jax.experimental import pallas as pl
from jax.experimental.pallas import tpu as pltpu
import jax.experimental.pallas.tpu_sc as plsc

NC, NS, L = 2, 16, 16
NW = NC * NS
N_P = 10000
N_A = 10000
D = 128
E = 320000
E_LABEL = 100000

CH = 128
CPT = 160
E_PAD = NS * CPT * CH
DUMMY = N_A
ACC_R = N_A + 8
APW = 624
TAIL = N_A - NS * APW
DH = D // NC
G = CPT
HIST_R = N_A + L

LCH = 26
EL_PAD = NW * LCH * CH


def _zero_f32(ref, rows, cols):
    def body(t, _):
        r = t // (cols // L)
        c = (t % (cols // L)) * L
        ref[r, pl.ds(c, L)] = jnp.zeros((L,), jnp.float32)
        return 0
    lax.fori_loop(0, rows * (cols // L), body, 0)


def _scatter_body(eidx_hbm, px_hbm, sum_hbm, cnt_hbm,
                  eidx_v, aid_w, pid_w, rows_v, hist_v,
                  acc_sh, px_sh, gsem, ssem):
    cid = lax.axis_index("c")
    sid = lax.axis_index("s")
    wid = cid * NS + sid

    _zero_f32(rows_v.at[0], CH, DH)
    def zh(t, _):
        hist_v[pl.ds(t * L, L)] = jnp.zeros((L,), jnp.float32)
        return 0
    lax.fori_loop(0, HIST_R // L, zh, 0)
    r0 = sid * APW
    nt = APW - (APW // CH) * CH
    for k in range(APW // CH):
        pltpu.sync_copy(rows_v.at[0], acc_sh.at[pl.ds(r0 + k * CH, CH)])
    pltpu.sync_copy(rows_v.at[0].at[pl.ds(0, nt)],
                    acc_sh.at[pl.ds(r0 + (APW // CH) * CH, nt)])
    pltpu.sync_copy(px_hbm.at[cid, pl.ds(r0, APW)], px_sh.at[pl.ds(r0, APW)])
    @pl.when(sid == NS - 1)
    def _():
        pltpu.sync_copy(rows_v.at[0].at[pl.ds(0, TAIL)],
                        acc_sh.at[pl.ds(NS * APW, TAIL)])
        pltpu.sync_copy(px_hbm.at[cid, pl.ds(NS * APW, TAIL)],
                        px_sh.at[pl.ds(NS * APW, TAIL)])
    pltpu.sync_copy(eidx_hbm.at[sid], eidx_v)
    plsc.subcore_barrier()

    vone = jnp.ones((L,), jnp.float32)
    m16 = jnp.full((L,), 0xFFFF, jnp.int32)

    def unpack_ids(g, p):
        for k2 in range(CH // L):
            pk = eidx_v[g, pl.ds(k2 * L, L)]
            pid_w[p, pl.ds(k2 * L, L)] = jnp.bitwise_and(pk, m16)
            aid_w[p, pl.ds(k2 * L, L)] = lax.shift_right_logical(pk, 16)

    def issue_gather(p):
        pltpu.async_copy(px_sh.at[pid_w.at[p]], rows_v.at[p], gsem)

    def wait_gather(p):
        pltpu.make_async_copy(px_sh.at[pid_w.at[p]],
                              rows_v.at[p], gsem).wait()

    def issue_scatter(p):
        pltpu.async_copy(rows_v.at[p],
                         acc_sh.at[aid_w.at[p]], ssem, add=True)

    def wait_scatter(p):
        pltpu.make_async_copy(rows_v.at[p],
                              acc_sh.at[aid_w.at[p]], ssem).wait()

    def histogram(p):
        for k2 in range(CH // L):
            idx = aid_w[p, pl.ds(k2 * L, L)]
            plsc.addupdate_scatter(hist_v, [idx], vone)

    unpack_ids(0, 0)
    issue_gather(0)

    def super_group(t, _):
        for p in range(2):
            g = 2 * t + p
            wait_gather(p)
            issue_scatter(p)
            @pl.when(g + 1 < G)
            def _():
                unpack_ids(g + 1, 1 - p)
                issue_gather(1 - p)
            histogram(p)
            wait_scatter(p)
        return 0

    lax.fori_loop(0, G // 2, super_group, 0)

    plsc.subcore_barrier()

    pltpu.sync_copy(acc_sh.at[pl.ds(r0, APW)], sum_hbm.at[cid, pl.ds(r0, APW)])
    @pl.when(sid == NS - 1)
    def _():
        pltpu.sync_copy(acc_sh.at[pl.ds(NS * APW, TAIL)],
                        sum_hbm.at[cid, pl.ds(NS * APW, TAIL)])
    pltpu.sync_copy(hist_v.at[pl.ds(0, N_A)], cnt_hbm.at[wid])


@functools.cache
def _scatter_call():
    mesh = plsc.VectorSubcoreMesh(
        core_axis_name="c", subcore_axis_name="s",
        num_cores=NC, num_subcores=NS)
    return pl.kernel(
        _scatter_body,
        out_type=(
            jax.ShapeDtypeStruct((NC, N_A, DH), jnp.float32),
            jax.ShapeDtypeStruct((NW, N_A), jnp.float32),
        ),
        mesh=mesh,
        compiler_params=pltpu.CompilerParams(
            use_tc_tiling_on_sc=False, needs_layout_passes=False),
        scratch_types=[
            pltpu.VMEM((G, CH), jnp.int32),
            pltpu.VMEM((2, CH), jnp.int32),
            pltpu.VMEM((2, CH), jnp.int32),
            pltpu.VMEM((2, CH, DH), jnp.float32),
            pltpu.VMEM((HIST_R,), jnp.float32),
            pltpu.VMEM_SHARED((ACC_R, DH), jnp.float32),
            pltpu.VMEM_SHARED((N_P, DH), jnp.float32),
            pltpu.SemaphoreType.DMA,
            pltpu.SemaphoreType.DMA,
        ],
    )


def _dense_body(px_ref, s_ref, c_ref, wp_ref, bp_ref, wa_ref, ba_ref,
                ph_ref, ah_ref):
    dn = (((1,), (1,)), ((), ()))
    wp = wp_ref[...]
    bp = bp_ref[...]
    px = px_ref[...]
    ph = lax.dot_general(
        px, wp, dn, precision=lax.Precision.HIGHEST,
        preferred_element_type=jnp.float32) + bp
    ph_ref[...] = ph.astype(jnp.bfloat16)
    s = jnp.concatenate([s_ref[0], s_ref[1]], axis=1)
    cnt = 0.5 * jnp.sum(c_ref[...], axis=1)[:, None]
    mean = s / jnp.maximum(cnt, 1.0)
    t = lax.dot_general(mean, wp, dn, precision=lax.Precision.HIGHEST,
                        preferred_element_type=jnp.float32)
    am = t + bp * (cnt > 0.0).astype(jnp.float32)
    ah = lax.dot_general(
        am, wa_ref[...], dn, precision=lax.Precision.HIGHEST,
        preferred_element_type=jnp.float32) + ba_ref[...]
    ah_ref[...] = ah.astype(jnp.bfloat16)


_BLK = 1000

_dense_call = pl.pallas_call(
    _dense_body,
    grid=(N_P // _BLK,),
    in_specs=[
        pl.BlockSpec((_BLK, D), lambda i: (i, 0)),
        pl.BlockSpec((NC, _BLK, DH), lambda i: (0, i, 0)),
        pl.BlockSpec((_BLK, NW), lambda i: (i, 0)),
        pl.BlockSpec((D, D), lambda i: (0, 0)),
        pl.BlockSpec((1, D), lambda i: (0, 0)),
        pl.BlockSpec((D, D), lambda i: (0, 0)),
        pl.BlockSpec((1, D), lambda i: (0, 0)),
    ],
    out_specs=[
        pl.BlockSpec((_BLK, D), lambda i: (i, 0)),
        pl.BlockSpec((_BLK, D), lambda i: (i, 0)),
    ],
    out_shape=[
        jax.ShapeDtypeStruct((N_P, D), jnp.bfloat16),
        jax.ShapeDtypeStruct((N_A, D), jnp.bfloat16),
    ],
)


def _pred_body(row_hbm, col_hbm, ah_hbm, ph_hbm, pred_hbm,
               r_v, c_v, ar_v, pr_v, out_v, tmp_v, ah_sh, ph_sh, sem):
    cid = lax.axis_index("c")
    sid = lax.axis_index("s")
    wid = cid * NS + sid

    r0 = sid * APW
    pltpu.sync_copy(ah_hbm.at[pl.ds(r0, APW)], ah_sh.at[pl.ds(r0, APW)])
    pltpu.sync_copy(ph_hbm.at[pl.ds(r0, APW)], ph_sh.at[pl.ds(r0, APW)])
    @pl.when(sid == NS - 1)
    def _():
        pltpu.sync_copy(ah_hbm.at[pl.ds(NS * APW, TAIL)],
                        ah_sh.at[pl.ds(NS * APW, TAIL)])
        pltpu.sync_copy(ph_hbm.at[pl.ds(NS * APW, TAIL)],
                        ph_sh.at[pl.ds(NS * APW, TAIL)])

    pltpu.sync_copy(row_hbm.at[wid], r_v)
    pltpu.sync_copy(col_hbm.at[wid], c_v)
    plsc.subcore_barrier()

    lane = lax.iota(jnp.int32, L)

    def issue(ci, b):
        pltpu.async_copy(ah_sh.at[r_v.at[ci]], ar_v.at[b], sem)
        pltpu.async_copy(ph_sh.at[c_v.at[ci]], pr_v.at[b], sem)

    def wait(ci, b):
        pltpu.make_async_copy(ah_sh.at[r_v.at[ci]], ar_v.at[b], sem).wait()
        pltpu.make_async_copy(ph_sh.at[c_v.at[ci]], pr_v.at[b], sem).wait()

    def compute(ci, b):
        def gbody(g, _):
            for u in range(L):
                e = g * L + u
                acc = jnp.zeros((L,), jnp.float32)
                for j in range(D // (2 * L)):
                    a2 = ar_v[b, e, pl.ds(j * 2 * L, 2 * L)]
                    p2 = pr_v[b, e, pl.ds(j * 2 * L, 2 * L)]
                    a0, a1 = plsc.unpack(a2, format=plsc.PackFormat.INTERLEAVED)
                    p0, p1 = plsc.unpack(p2, format=plsc.PackFormat.INTERLEAVED)
                    acc = acc + a0 * p0 + a1 * p1
                tmp_v[u, pl.ds(0, L)] = acc
            res = plsc.load_gather(tmp_v, [lane, jnp.zeros((L,), jnp.int32)])
            for c in range(1, L):
                res = res + plsc.load_gather(
                    tmp_v, [lane, jnp.full((L,), c, jnp.int32)])
            out_v[ci, pl.ds(g * L, L)] = res
            return 0
        lax.fori_loop(0, CH // L, gbody, 0)

    issue(0, 0)

    def super_chunk(t, _):
        for b in range(2):
            ci = 2 * t + b
            wait(ci, b)
            @pl.when(ci + 1 < LCH)
            def _():
                issue(ci + 1, 1 - b)
            compute(ci, b)
        return 0

    lax.fori_loop(0, LCH // 2, super_chunk, 0)
    pltpu.sync_copy(out_v, pred_hbm.at[wid])


@functools.cache
def _pred_call():
    mesh = plsc.VectorSubcoreMesh(
        core_axis_name="c", subcore_axis_name="s",
        num_cores=NC, num_subcores=NS)
    return pl.kernel(
        _pred_body,
        out_type=jax.ShapeDtypeStruct((NW, LCH, CH), jnp.float32),
        mesh=mesh,
        compiler_params=pltpu.CompilerParams(
            use_tc_tiling_on_sc=False, needs_layout_passes=False),
        scratch_types=[
            pltpu.VMEM((LCH, CH), jnp.int32),
            pltpu.VMEM((LCH, CH), jnp.int32),
            pltpu.VMEM((2, CH, D), jnp.bfloat16),
            pltpu.VMEM((2, CH, D), jnp.bfloat16),
            pltpu.VMEM((LCH, CH), jnp.float32),
            pltpu.VMEM((L, L), jnp.float32),
            pltpu.VMEM_SHARED((N_A, D), jnp.bfloat16),
            pltpu.VMEM_SHARED((N_A, D), jnp.bfloat16),
            pltpu.SemaphoreType.DMA,
        ],
    )


def kernel(paper_x, edge_index, edge_label_index, Wp, bp, Wa, ba):
    epad = E_PAD - E
    aid = jnp.concatenate(
        [edge_index[0], jnp.full((epad,), DUMMY, jnp.int32)])
    pid = jnp.concatenate(
        [edge_index[1], jnp.zeros((epad,), jnp.int32)])
    eidx = jnp.bitwise_or(jnp.left_shift(aid, 16), pid).reshape(NS, G, CH)
    px_halves = paper_x.reshape(N_P, NC, DH).transpose(1, 0, 2)
    sums, cnts = _scatter_call()(eidx, px_halves)
    ph, ah = _dense_call(paper_x, sums, cnts.T, Wp, bp.reshape(1, D),
                         Wa, ba.reshape(1, D))
    pad = EL_PAD - E_LABEL
    row = jnp.concatenate(
        [edge_label_index[0], jnp.zeros((pad,), jnp.int32)]
    ).reshape(NW, LCH, CH)
    col = jnp.concatenate(
        [edge_label_index[1], jnp.zeros((pad,), jnp.int32)]
    ).reshape(NW, LCH, CH)
    pred = _pred_call()(row, col, ah, ph)
    return pred.reshape(-1)[:E_LABEL]


# speedup vs baseline: 2.4468x; 1.0116x over previous
import functools

import jax
import jax.numpy as jnp
from jax import lax
from jax.experimental import pallas as pl
from jax.experimental.pallas import tpu as pltpu
import jax.experimental.pallas.tpu_sc as plsc

NC, NS, L = 2, 16, 16
NW = NC * NS
N_P = 10000
N_A = 10000
D = 128
E = 320000
E_LABEL = 100000

CH = 128
CPT = 162
E_PAD = NS * CPT * CH
DUMMY = N_A
ACC_R = N_A + 8
APW = 624
TAIL = N_A - NS * APW
DH = D // NC
G = CPT
PH = G // 2
NB = 3
HIST_R = N_A + L

LCH = 26
EL_PAD = NW * LCH * CH


def _zero_f32(ref, rows, cols):
    def body(t, _):
        r = t // (cols // L)
        c = (t % (cols // L)) * L
        ref[r, pl.ds(c, L)] = jnp.zeros((L,), jnp.float32)
        return 0
    lax.fori_loop(0, rows * (cols // L), body, 0)


def _scatter_body(eidx_hbm, px_hbm, sum_hbm, cnt_hbm,
                  eidx_v, aid_w, pid_w, rows_v, hist_v,
                  acc_sh, px_sh, gsem, ssem):
    cid = lax.axis_index("c")
    sid = lax.axis_index("s")
    wid = cid * NS + sid

    _zero_f32(rows_v.at[0], CH, DH)
    def zh(t, _):
        hist_v[pl.ds(t * L, L)] = jnp.zeros((L,), jnp.float32)
        return 0
    lax.fori_loop(0, HIST_R // L, zh, 0)
    r0 = sid * APW
    nt = APW - (APW // CH) * CH
    for k in range(APW // CH):
        pltpu.sync_copy(rows_v.at[0], acc_sh.at[pl.ds(r0 + k * CH, CH)])
    pltpu.sync_copy(rows_v.at[0].at[pl.ds(0, nt)],
                    acc_sh.at[pl.ds(r0 + (APW // CH) * CH, nt)])
    pltpu.sync_copy(px_hbm.at[cid, pl.ds(r0, APW)], px_sh.at[pl.ds(r0, APW)])
    @pl.when(sid == NS - 1)
    def _():
        pltpu.sync_copy(rows_v.at[0].at[pl.ds(0, TAIL)],
                        acc_sh.at[pl.ds(NS * APW, TAIL)])
        pltpu.sync_copy(px_hbm.at[cid, pl.ds(NS * APW, TAIL)],
                        px_sh.at[pl.ds(NS * APW, TAIL)])
    plsc.subcore_barrier()

    vone = jnp.ones((L,), jnp.float32)
    m16 = jnp.full((L,), 0xFFFF, jnp.int32)

    def unpack_ids(g, p):
        for k2 in range(CH // L):
            pk = eidx_v[g, pl.ds(k2 * L, L)]
            pid_w[p, pl.ds(k2 * L, L)] = jnp.bitwise_and(pk, m16)
            aid_w[p, pl.ds(k2 * L, L)] = lax.shift_right_logical(pk, 16)

    def issue_gather(p):
        pltpu.async_copy(px_sh.at[pid_w.at[p]], rows_v.at[p], gsem)

    def wait_gather(p):
        pltpu.make_async_copy(px_sh.at[pid_w.at[p]],
                              rows_v.at[p], gsem).wait()

    def issue_scatter(p):
        pltpu.async_copy(rows_v.at[p],
                         acc_sh.at[aid_w.at[p]], ssem, add=True)

    def wait_scatter(p):
        pltpu.make_async_copy(rows_v.at[p],
                              acc_sh.at[aid_w.at[p]], ssem).wait()

    def histogram(p):
        for k2 in range(CH // L):
            idx = aid_w[p, pl.ds(k2 * L, L)]
            plsc.addupdate_scatter(hist_v, [idx], vone)

    for phase in range(2):
        pltpu.sync_copy(eidx_hbm.at[sid, pl.ds(phase * PH, PH)], eidx_v)
        unpack_ids(0, 0)
        issue_gather(0)
        unpack_ids(1, 1)
        issue_gather(1)

        def super_group(t, _):
            for p in range(NB):
                g = NB * t + p
                wait_gather(p)
                issue_scatter(p)
                @pl.when(g + 2 < PH)
                def _():
                    unpack_ids(g + 2, (p + 2) % NB)
                    issue_gather((p + 2) % NB)
                histogram(p)
                wait_scatter(p)
            return 0

        lax.fori_loop(0, PH // NB, super_group, 0)

    plsc.subcore_barrier()

    pltpu.sync_copy(acc_sh.at[pl.ds(r0, APW)], sum_hbm.at[cid, pl.ds(r0, APW)])
    @pl.when(sid == NS - 1)
    def _():
        pltpu.sync_copy(acc_sh.at[pl.ds(NS * APW, TAIL)],
                        sum_hbm.at[cid, pl.ds(NS * APW, TAIL)])
    pltpu.sync_copy(hist_v.at[pl.ds(0, N_A)], cnt_hbm.at[wid])


@functools.cache
def _scatter_call():
    mesh = plsc.VectorSubcoreMesh(
        core_axis_name="c", subcore_axis_name="s",
        num_cores=NC, num_subcores=NS)
    return pl.kernel(
        _scatter_body,
        out_type=(
            jax.ShapeDtypeStruct((NC, N_A, DH), jnp.float32),
            jax.ShapeDtypeStruct((NW, N_A), jnp.float32),
        ),
        mesh=mesh,
        compiler_params=pltpu.CompilerParams(
            use_tc_tiling_on_sc=False, needs_layout_passes=False),
        scratch_types=[
            pltpu.VMEM((PH, CH), jnp.int32),
            pltpu.VMEM((NB, CH), jnp.int32),
            pltpu.VMEM((NB, CH), jnp.int32),
            pltpu.VMEM((NB, CH, DH), jnp.float32),
            pltpu.VMEM((HIST_R,), jnp.float32),
            pltpu.VMEM_SHARED((ACC_R, DH), jnp.float32),
            pltpu.VMEM_SHARED((N_P, DH), jnp.float32),
            pltpu.SemaphoreType.DMA,
            pltpu.SemaphoreType.DMA,
        ],
    )


def _dense_body(px_ref, s_ref, c_ref, wp_ref, bp_ref, wa_ref, ba_ref,
                ph_ref, ah_ref):
    dn = (((1,), (1,)), ((), ()))
    wp = wp_ref[...]
    bp = bp_ref[...]
    px = px_ref[...]
    ph = lax.dot_general(
        px, wp, dn, precision=lax.Precision.HIGHEST,
        preferred_element_type=jnp.float32) + bp
    ph_ref[...] = ph.astype(jnp.bfloat16)
    s = jnp.concatenate([s_ref[0], s_ref[1]], axis=1)
    cnt = 0.5 * jnp.sum(c_ref[...], axis=1)[:, None]
    mean = s / jnp.maximum(cnt, 1.0)
    t = lax.dot_general(mean, wp, dn, precision=lax.Precision.HIGHEST,
                        preferred_element_type=jnp.float32)
    am = t + bp * (cnt > 0.0).astype(jnp.float32)
    ah = lax.dot_general(
        am, wa_ref[...], dn, precision=lax.Precision.HIGHEST,
        preferred_element_type=jnp.float32) + ba_ref[...]
    ah_ref[...] = ah.astype(jnp.bfloat16)


_BLK = 1000

_dense_call = pl.pallas_call(
    _dense_body,
    grid=(N_P // _BLK,),
    in_specs=[
        pl.BlockSpec((_BLK, D), lambda i: (i, 0)),
        pl.BlockSpec((NC, _BLK, DH), lambda i: (0, i, 0)),
        pl.BlockSpec((_BLK, NW), lambda i: (i, 0)),
        pl.BlockSpec((D, D), lambda i: (0, 0)),
        pl.BlockSpec((1, D), lambda i: (0, 0)),
        pl.BlockSpec((D, D), lambda i: (0, 0)),
        pl.BlockSpec((1, D), lambda i: (0, 0)),
    ],
    out_specs=[
        pl.BlockSpec((_BLK, D), lambda i: (i, 0)),
        pl.BlockSpec((_BLK, D), lambda i: (i, 0)),
    ],
    out_shape=[
        jax.ShapeDtypeStruct((N_P, D), jnp.bfloat16),
        jax.ShapeDtypeStruct((N_A, D), jnp.bfloat16),
    ],
)


def _pred_body(row_hbm, col_hbm, ah_hbm, ph_hbm, pred_hbm,
               r_v, c_v, ar_v, pr_v, out_v, tmp_v, ah_sh, ph_sh, sem):
    cid = lax.axis_index("c")
    sid = lax.axis_index("s")
    wid = cid * NS + sid

    r0 = sid * APW
    pltpu.sync_copy(ah_hbm.at[pl.ds(r0, APW)], ah_sh.at[pl.ds(r0, APW)])
    pltpu.sync_copy(ph_hbm.at[pl.ds(r0, APW)], ph_sh.at[pl.ds(r0, APW)])
    @pl.when(sid == NS - 1)
    def _():
        pltpu.sync_copy(ah_hbm.at[pl.ds(NS * APW, TAIL)],
                        ah_sh.at[pl.ds(NS * APW, TAIL)])
        pltpu.sync_copy(ph_hbm.at[pl.ds(NS * APW, TAIL)],
                        ph_sh.at[pl.ds(NS * APW, TAIL)])

    pltpu.sync_copy(row_hbm.at[wid], r_v)
    pltpu.sync_copy(col_hbm.at[wid], c_v)
    plsc.subcore_barrier()

    lane = lax.iota(jnp.int32, L)

    def issue(ci, b):
        pltpu.async_copy(ah_sh.at[r_v.at[ci]], ar_v.at[b], sem)
        pltpu.async_copy(ph_sh.at[c_v.at[ci]], pr_v.at[b], sem)

    def wait(ci, b):
        pltpu.make_async_copy(ah_sh.at[r_v.at[ci]], ar_v.at[b], sem).wait()
        pltpu.make_async_copy(ph_sh.at[c_v.at[ci]], pr_v.at[b], sem).wait()

    def compute(ci, b):
        def gbody(g, _):
            for u in range(L):
                e = g * L + u
                acc = jnp.zeros((L,), jnp.float32)
                for j in range(D // (2 * L)):
                    a2 = ar_v[b, e, pl.ds(j * 2 * L, 2 * L)]
                    p2 = pr_v[b, e, pl.ds(j * 2 * L, 2 * L)]
                    a0, a1 = plsc.unpack(a2, format=plsc.PackFormat.INTERLEAVED)
                    p0, p1 = plsc.unpack(p2, format=plsc.PackFormat.INTERLEAVED)
                    acc = acc + a0 * p0 + a1 * p1
                tmp_v[u, pl.ds(0, L)] = acc
            res = plsc.load_gather(tmp_v, [lane, jnp.zeros((L,), jnp.int32)])
            for c in range(1, L):
                res = res + plsc.load_gather(
                    tmp_v, [lane, jnp.full((L,), c, jnp.int32)])
            out_v[ci, pl.ds(g * L, L)] = res
            return 0
        lax.fori_loop(0, CH // L, gbody, 0)

    issue(0, 0)

    def super_chunk(t, _):
        for b in range(2):
            ci = 2 * t + b
            wait(ci, b)
            @pl.when(ci + 1 < LCH)
            def _():
                issue(ci + 1, 1 - b)
            compute(ci, b)
        return 0

    lax.fori_loop(0, LCH // 2, super_chunk, 0)
    pltpu.sync_copy(out_v, pred_hbm.at[wid])


@functools.cache
def _pred_call():
    mesh = plsc.VectorSubcoreMesh(
        core_axis_name="c", subcore_axis_name="s",
        num_cores=NC, num_subcores=NS)
    return pl.kernel(
        _pred_body,
        out_type=jax.ShapeDtypeStruct((NW, LCH, CH), jnp.float32),
        mesh=mesh,
        compiler_params=pltpu.CompilerParams(
            use_tc_tiling_on_sc=False, needs_layout_passes=False),
        scratch_types=[
            pltpu.VMEM((LCH, CH), jnp.int32),
            pltpu.VMEM((LCH, CH), jnp.int32),
            pltpu.VMEM((2, CH, D), jnp.bfloat16),
            pltpu.VMEM((2, CH, D), jnp.bfloat16),
            pltpu.VMEM((LCH, CH), jnp.float32),
            pltpu.VMEM((L, L), jnp.float32),
            pltpu.VMEM_SHARED((N_A, D), jnp.bfloat16),
            pltpu.VMEM_SHARED((N_A, D), jnp.bfloat16),
            pltpu.SemaphoreType.DMA,
        ],
    )


def kernel(paper_x, edge_index, edge_label_index, Wp, bp, Wa, ba):
    epad = E_PAD - E
    aid = jnp.concatenate(
        [edge_index[0], jnp.full((epad,), DUMMY, jnp.int32)])
    pid = jnp.concatenate(
        [edge_index[1], jnp.zeros((epad,), jnp.int32)])
    eidx = jnp.bitwise_or(jnp.left_shift(aid, 16), pid).reshape(NS, G, CH)
    px_halves = paper_x.reshape(N_P, NC, DH).transpose(1, 0, 2)
    sums, cnts = _scatter_call()(eidx, px_halves)
    ph, ah = _dense_call(paper_x, sums, cnts.T, Wp, bp.reshape(1, D),
                         Wa, ba.reshape(1, D))
    pad = EL_PAD - E_LABEL
    row = jnp.concatenate(
        [edge_label_index[0], jnp.zeros((pad,), jnp.int32)]
    ).reshape(NW, LCH, CH)
    col = jnp.concatenate(
        [edge_label_index[1], jnp.zeros((pad,), jnp.int32)]
    ).reshape(NW, LCH, CH)
    pred = _pred_call()(row, col, ah, ph)
    return pred.reshape(-1)[:E_LABEL]
